# baseline (speedup 1.0000x reference)
import functools

import jax
import jax.numpy as jnp
from jax import lax
from jax.experimental import pallas as pl
from jax.experimental.pallas import tpu as pltpu
from jax.experimental.pallas import tpu_sc as plsc

B = 16
S = 16
V = 100000
L = 16
VCHUNKS = V // L


def _argmax_rows_body(table_hbm, seq_hbm, out_hbm, sl_v, row_v, outv, sem):
    c = lax.axis_index("c")
    s = lax.axis_index("s")
    b = c * 8 + s

    @pl.when(s < 8)
    def _():
        pltpu.sync_copy(seq_hbm, sl_v)
        iota = lax.iota(jnp.int32, L)
        slb = plsc.load_gather(sl_v.at[:], [jnp.full((L,), b, jnp.int32)])
        rowid = b * S + slb[0] - 1
        pltpu.async_copy(table_hbm.at[pl.ds(rowid * V, V)], row_v,
                         sem).wait()

        def body(i, carry):
            cm, ci, base = carry
            v = row_v[pl.ds(i * L, L)]
            m = v > cm
            cm = jnp.where(m, v, cm)
            ci = jnp.where(m, base, ci)
            return cm, ci, base + L

        cm0 = jnp.full((L,), -jnp.inf, jnp.float32)
        ci0 = jnp.zeros((L,), jnp.int32)
        cm, ci, _ = lax.fori_loop(0, VCHUNKS, body, (cm0, ci0, iota))

        for shift in (8, 4, 2, 1):
            perm = iota ^ shift
            om = cm.at[perm].get(mode="promise_in_bounds")
            oi = ci.at[perm].get(mode="promise_in_bounds")
            better = (om > cm) | ((om == cm) & (oi < ci))
            cm = jnp.where(better, om, cm)
            ci = jnp.where(better, oi, ci)
        outv[...] = ci
        pltpu.sync_copy(outv, out_hbm.at[b])


def kernel(logits, seq_lens):
    table = logits.reshape(B * S * V)
    sl = seq_lens.astype(jnp.int32)
    mesh = plsc.VectorSubcoreMesh(core_axis_name="c", subcore_axis_name="s")
    run = functools.partial(
        pl.kernel,
        mesh=mesh,
        out_type=jax.ShapeDtypeStruct((B, L), jnp.int32),
        scratch_types=[
            pltpu.VMEM((L,), jnp.int32),
            pltpu.VMEM((V,), jnp.float32),
            pltpu.VMEM((L,), jnp.int32),
            pltpu.SemaphoreType.DMA,
        ],
        compiler_params=pltpu.CompilerParams(needs_layout_passes=False),
    )(_argmax_rows_body)
    out = run(table, sl)
    return out[:, :1]


# speedup vs baseline: 1.1010x; 1.1010x over previous
import functools

import jax
import jax.numpy as jnp
from jax import lax
from jax.experimental import pallas as pl
from jax.experimental.pallas import tpu as pltpu
from jax.experimental.pallas import tpu_sc as plsc

B = 16
S = 16
V = 100000
L = 16
VCHUNKS = V // L


def _argmax_rows_body(table_hbm, seq_hbm, out_hbm, sl_v, row_v, outv, sem):
    c = lax.axis_index("c")
    s = lax.axis_index("s")
    b = c * 8 + s

    @pl.when(s < 8)
    def _():
        pltpu.sync_copy(seq_hbm, sl_v)
        iota = lax.iota(jnp.int32, L)
        slb = plsc.load_gather(sl_v.at[:], [jnp.full((L,), b, jnp.int32)])
        rowid = b * S + slb[0] - 1
        pltpu.async_copy(table_hbm.at[pl.ds(rowid * V, V)], row_v,
                         sem).wait()

        def body(i, carry):
            cm, ci, base = carry
            v = row_v[pl.ds(i * L, L)]
            m = v > cm
            cm = jnp.where(m, v, cm)
            ci = jnp.where(m, base, ci)
            return cm, ci, base + L

        cm0 = jnp.full((L,), -jnp.inf, jnp.float32)
        ci0 = jnp.zeros((L,), jnp.int32)
        cm, ci, _ = lax.fori_loop(0, VCHUNKS, body, (cm0, ci0, iota),
                                  unroll=25)

        for shift in (8, 4, 2, 1):
            perm = iota ^ shift
            om = cm.at[perm].get(mode="promise_in_bounds")
            oi = ci.at[perm].get(mode="promise_in_bounds")
            better = (om > cm) | ((om == cm) & (oi < ci))
            cm = jnp.where(better, om, cm)
            ci = jnp.where(better, oi, ci)
        outv[...] = ci
        pltpu.sync_copy(outv, out_hbm.at[b])


def kernel(logits, seq_lens):
    table = logits.reshape(B * S * V)
    sl = seq_lens.astype(jnp.int32)
    mesh = plsc.VectorSubcoreMesh(core_axis_name="c", subcore_axis_name="s")
    run = functools.partial(
        pl.kernel,
        mesh=mesh,
        out_type=jax.ShapeDtypeStruct((B, L), jnp.int32),
        scratch_types=[
            pltpu.VMEM((L,), jnp.int32),
            pltpu.VMEM((V,), jnp.float32),
            pltpu.VMEM((L,), jnp.int32),
            pltpu.SemaphoreType.DMA,
        ],
        compiler_params=pltpu.CompilerParams(needs_layout_passes=False),
    )(_argmax_rows_body)
    out = run(table, sl)
    return out[:, :1]


# speedup vs baseline: 5.7580x; 5.2296x over previous
import functools

import jax
import jax.numpy as jnp
from jax import lax
from jax.experimental import pallas as pl
from jax.experimental.pallas import tpu as pltpu
from jax.experimental.pallas import tpu_sc as plsc

B = 16
S = 16
V = 100000
L = 16
VCHUNKS = V // L


def _argmax_rows_body(table_hbm, seq_hbm, out_hbm, sl_v, row_v, outv, sem):
    c = lax.axis_index("c")
    s = lax.axis_index("s")
    b = c * 8 + s

    @pl.when(s < 8)
    def _():
        pltpu.sync_copy(seq_hbm, sl_v)
        iota = lax.iota(jnp.int32, L)
        slb = plsc.load_gather(sl_v.at[:], [jnp.full((L,), b, jnp.int32)])
        r = slb[0] - 1
        pltpu.async_copy(table_hbm.at[b, r], row_v, sem).wait()

        def body(i, carry):
            cm, ci, base = carry
            v = row_v[pl.ds(i * L, L)]
            m = v > cm
            cm = jnp.where(m, v, cm)
            ci = jnp.where(m, base, ci)
            return cm, ci, base + L

        cm0 = jnp.full((L,), -jnp.inf, jnp.float32)
        ci0 = jnp.zeros((L,), jnp.int32)
        cm, ci, _ = lax.fori_loop(0, VCHUNKS, body, (cm0, ci0, iota),
                                  unroll=25)

        for shift in (8, 4, 2, 1):
            perm = iota ^ shift
            om = cm.at[perm].get(mode="promise_in_bounds")
            oi = ci.at[perm].get(mode="promise_in_bounds")
            better = (om > cm) | ((om == cm) & (oi < ci))
            cm = jnp.where(better, om, cm)
            ci = jnp.where(better, oi, ci)
        outv[...] = ci
        pltpu.sync_copy(outv, out_hbm.at[b])


def kernel(logits, seq_lens):
    table = logits
    sl = seq_lens.astype(jnp.int32)
    mesh = plsc.VectorSubcoreMesh(core_axis_name="c", subcore_axis_name="s")
    run = functools.partial(
        pl.kernel,
        mesh=mesh,
        out_type=jax.ShapeDtypeStruct((B, L), jnp.int32),
        scratch_types=[
            pltpu.VMEM((L,), jnp.int32),
            pltpu.VMEM((V,), jnp.float32),
            pltpu.VMEM((L,), jnp.int32),
            pltpu.SemaphoreType.DMA,
        ],
        compiler_params=pltpu.CompilerParams(needs_layout_passes=False),
    )(_argmax_rows_body)
    out = run(table, sl)
    return out[:, :1]


# speedup vs baseline: 5.7605x; 1.0004x over previous
import functools

import jax
import jax.numpy as jnp
from jax import lax
from jax.experimental import pallas as pl
from jax.experimental.pallas import tpu as pltpu
from jax.experimental.pallas import tpu_sc as plsc

B = 16
S = 16
V = 100000
L = 16
VCHUNKS = V // L


CH = 12800
VALN = 99968
TAILN = V - VALN
_CHUNKS = [(k * CH, CH) for k in range(7)] + [(7 * CH, VALN - 7 * CH)]


def _argmax_rows_body(table_hbm, tail_hbm, seq_hbm, out_hbm, sl_v, buf_a,
                      buf_b, tail_v, outv, sem_a, sem_b, sem_t):
    c = lax.axis_index("c")
    s = lax.axis_index("s")
    b = c * 8 + s

    @pl.when(s < 8)
    def _():
        pltpu.sync_copy(seq_hbm, sl_v)
        iota = lax.iota(jnp.int32, L)
        slb = plsc.load_gather(sl_v.at[:], [jnp.full((L,), b, jnp.int32)])
        r = slb[0] - 1

        bufs = [buf_a, buf_b]
        sems = [sem_a, sem_b]

        def start(k):
            off, ln = _CHUNKS[k]
            return pltpu.async_copy(
                table_hbm.at[b, r, pl.ds(off, ln)],
                bufs[k % 2].at[pl.ds(0, ln)], sems[k % 2])

        def body_for(buf):
            def body(i, carry):
                cm, ci, base = carry
                v = buf[pl.ds(i * L, L)]
                m = v > cm
                cm = jnp.where(m, v, cm)
                ci = jnp.where(m, base, ci)
                return cm, ci, base + L
            return body

        cm = jnp.full((L,), -jnp.inf, jnp.float32)
        ci = jnp.zeros((L,), jnp.int32)
        cp = start(0)
        cp_t = pltpu.async_copy(tail_hbm.at[b, r], tail_v, sem_t)
        for k, (off, ln) in enumerate(_CHUNKS):
            cp.wait()
            if k + 1 < len(_CHUNKS):
                cp = start(k + 1)
            cm, ci, _ = lax.fori_loop(
                0, ln // L, body_for(bufs[k % 2]),
                (cm, ci, iota + off), unroll=25 if ln % (25 * L) == 0 else 8)
        cp_t.wait()
        for j in range(TAILN // L):
            v = tail_v[pl.ds(j * L, L)]
            base = iota + (VALN + j * L)
            m = v > cm
            cm = jnp.where(m, v, cm)
            ci = jnp.where(m, base, ci)

        for shift in (8, 4, 2, 1):
            perm = iota ^ shift
            om = cm.at[perm].get(mode="promise_in_bounds")
            oi = ci.at[perm].get(mode="promise_in_bounds")
            better = (om > cm) | ((om == cm) & (oi < ci))
            cm = jnp.where(better, om, cm)
            ci = jnp.where(better, oi, ci)
        outv[...] = ci
        pltpu.sync_copy(outv, out_hbm.at[b])


def kernel(logits, seq_lens):
    table = logits
    tail = lax.slice(logits, (0, 0, VALN), (B, S, V))
    sl = seq_lens.astype(jnp.int32)
    mesh = plsc.VectorSubcoreMesh(core_axis_name="c", subcore_axis_name="s")
    run = functools.partial(
        pl.kernel,
        mesh=mesh,
        out_type=jax.ShapeDtypeStruct((B, L), jnp.int32),
        scratch_types=[
            pltpu.VMEM((L,), jnp.int32),
            pltpu.VMEM((CH,), jnp.float32),
            pltpu.VMEM((CH,), jnp.float32),
            pltpu.VMEM((TAILN,), jnp.float32),
            pltpu.VMEM((L,), jnp.int32),
            pltpu.SemaphoreType.DMA,
            pltpu.SemaphoreType.DMA,
            pltpu.SemaphoreType.DMA,
        ],
        compiler_params=pltpu.CompilerParams(needs_layout_passes=False),
    )(_argmax_rows_body)
    out = run(table, tail, sl)
    return out[:, :1]
